# Initial kernel scaffold; baseline (speedup 1.0000x reference)
import jax
import jax.numpy as jnp
from jax.experimental import pallas as pl


def kernel(cosine, label):
    raise NotImplementedError("write your pallas kernel here")


# speedup vs baseline: 1.0299x; 1.0299x over previous
import jax
import jax.numpy as jnp
from jax.experimental import pallas as pl

S = 64.0
M = 0.4

_BLOCK_B = 256
_BLOCK_C = 2048


def _body(lab_ref, cos_ref, out_ref):
    j = pl.program_id(1)
    lab = lab_ref[:, 0]
    col0 = j * _BLOCK_C
    cols = col0 + jax.lax.broadcasted_iota(jnp.int32, (_BLOCK_B, _BLOCK_C), 1)
    mask = (cols == lab[:, None]).astype(jnp.float32)
    out_ref[...] = S * cos_ref[...] - (S * M) * mask


def kernel(cosine, label):
    B, C = cosine.shape
    lab2d = label.reshape(B, 1)
    grid = (B // _BLOCK_B, pl.cdiv(C, _BLOCK_C))
    return pl.pallas_call(
        _body,
        grid=grid,
        in_specs=[
            pl.BlockSpec((_BLOCK_B, 1), lambda i, j: (i, 0)),
            pl.BlockSpec((_BLOCK_B, _BLOCK_C), lambda i, j: (i, j)),
        ],
        out_specs=pl.BlockSpec((_BLOCK_B, _BLOCK_C), lambda i, j: (i, j)),
        out_shape=jax.ShapeDtypeStruct((B, C), cosine.dtype),
    )(lab2d, cosine)


# speedup vs baseline: 1.0633x; 1.0324x over previous
import jax
import jax.numpy as jnp
from jax.experimental import pallas as pl

S = 64.0
M = 0.4

_BLOCK_B = 512
_BLOCK_C = 4096


def _body(lab_ref, cos_ref, out_ref):
    j = pl.program_id(1)
    lab = lab_ref[:, 0]
    col0 = j * _BLOCK_C
    cols = col0 + jax.lax.broadcasted_iota(jnp.int32, (_BLOCK_B, _BLOCK_C), 1)
    mask = (cols == lab[:, None]).astype(jnp.float32)
    out_ref[...] = S * cos_ref[...] - (S * M) * mask


def kernel(cosine, label):
    B, C = cosine.shape
    lab2d = label.reshape(B, 1)
    grid = (B // _BLOCK_B, pl.cdiv(C, _BLOCK_C))
    return pl.pallas_call(
        _body,
        grid=grid,
        in_specs=[
            pl.BlockSpec((_BLOCK_B, 1), lambda i, j: (i, 0)),
            pl.BlockSpec((_BLOCK_B, _BLOCK_C), lambda i, j: (i, j)),
        ],
        out_specs=pl.BlockSpec((_BLOCK_B, _BLOCK_C), lambda i, j: (i, j)),
        out_shape=jax.ShapeDtypeStruct((B, C), cosine.dtype),
    )(lab2d, cosine)


# speedup vs baseline: 1.0636x; 1.0003x over previous
import jax
import jax.numpy as jnp
from jax.experimental import pallas as pl

S = 64.0
M = 0.4

_BLOCK_B = 1024
_BLOCK_C = 2048


def _body(lab_ref, cos_ref, out_ref):
    j = pl.program_id(1)
    lab = lab_ref[:, 0]
    col0 = j * _BLOCK_C
    cols = col0 + jax.lax.broadcasted_iota(jnp.int32, (_BLOCK_B, _BLOCK_C), 1)
    mask = (cols == lab[:, None]).astype(jnp.float32)
    out_ref[...] = S * cos_ref[...] - (S * M) * mask


def kernel(cosine, label):
    B, C = cosine.shape
    lab2d = label.reshape(B, 1)
    grid = (B // _BLOCK_B, pl.cdiv(C, _BLOCK_C))
    return pl.pallas_call(
        _body,
        grid=grid,
        in_specs=[
            pl.BlockSpec((_BLOCK_B, 1), lambda i, j: (i, 0)),
            pl.BlockSpec((_BLOCK_B, _BLOCK_C), lambda i, j: (i, j)),
        ],
        out_specs=pl.BlockSpec((_BLOCK_B, _BLOCK_C), lambda i, j: (i, j)),
        out_shape=jax.ShapeDtypeStruct((B, C), cosine.dtype),
    )(lab2d, cosine)
